# Initial kernel scaffold; baseline (speedup 1.0000x reference)
import jax
import jax.numpy as jnp
from jax.experimental import pallas as pl


def kernel(x, table):
    raise NotImplementedError("write your pallas kernel here")


# speedup vs baseline: 1.0944x; 1.0944x over previous
import functools

import jax
import jax.numpy as jnp
from jax import lax
from jax.experimental import pallas as pl
from jax.experimental.pallas import tpu as pltpu
from jax.experimental.pallas import tpu_sc as plsc

_BATCH = 16384
_HIST = 50
_EMBED = 32
_B = _BATCH * _HIST

_NC = 2
_NS = 16
_NW = _NC * _NS
_B_PER_W = _B // _NW

_CHUNK = 1024
_N_CHUNKS = _B_PER_W // _CHUNK


def _build():
    mesh = plsc.VectorSubcoreMesh(core_axis_name="c", subcore_axis_name="s")

    @functools.partial(
        pl.kernel,
        out_type=jax.ShapeDtypeStruct((_B, _EMBED), jnp.float32),
        mesh=mesh,
        scratch_types=[
            pltpu.VMEM((_CHUNK,), jnp.int32),
            pltpu.VMEM((_CHUNK, _EMBED), jnp.float32),
            pltpu.SemaphoreType.DMA,
        ],
        compiler_params=pltpu.CompilerParams(use_tc_tiling_on_sc=False),
    )
    def gather_kernel(x_hbm, table_hbm, out_hbm, idx_v, rows_v, sem):
        wid = lax.axis_index("s") * _NC + lax.axis_index("c")
        base = wid * _B_PER_W

        def body(i, carry):
            off = base + i * _CHUNK
            pltpu.sync_copy(x_hbm.at[pl.ds(off, _CHUNK)], idx_v)
            pltpu.async_copy(table_hbm.at[idx_v], rows_v, sem).wait()
            pltpu.sync_copy(rows_v, out_hbm.at[pl.ds(off, _CHUNK)])
            return carry

        lax.fori_loop(0, _N_CHUNKS, body, 0)

    return gather_kernel


_GATHER = _build()


@jax.jit
def kernel(x, table):
    flat = x.reshape(_B).astype(jnp.int32)
    out = _GATHER(flat, table)
    return out.reshape(_BATCH, _HIST, _EMBED)


# speedup vs baseline: 1.1129x; 1.0169x over previous
import functools

import jax
import jax.numpy as jnp
from jax import lax
from jax.experimental import pallas as pl
from jax.experimental.pallas import tpu as pltpu
from jax.experimental.pallas import tpu_sc as plsc

_BATCH = 16384
_HIST = 50
_EMBED = 32
_B = _BATCH * _HIST

_NC = 2
_NS = 16
_NW = _NC * _NS
_B_PER_W = _B // _NW

_CHUNK = 640
_N_CHUNKS = _B_PER_W // _CHUNK
_NBUF = 4


def _build():
    mesh = plsc.VectorSubcoreMesh(core_axis_name="c", subcore_axis_name="s")

    @functools.partial(
        pl.kernel,
        out_type=jax.ShapeDtypeStruct((_B, _EMBED), jnp.float32),
        mesh=mesh,
        scratch_types=[
            pltpu.VMEM((_N_CHUNKS, _CHUNK), jnp.int32),
            pltpu.VMEM((_NBUF, _CHUNK, _EMBED), jnp.float32),
            pltpu.SemaphoreType.DMA,
            pltpu.SemaphoreType.DMA,
        ],
        compiler_params=pltpu.CompilerParams(use_tc_tiling_on_sc=False),
    )
    def gather_kernel(x_hbm, table_hbm, out_hbm, idx_v, rows_v, gsem, ssem):
        wid = lax.axis_index("s") * _NC + lax.axis_index("c")
        base = wid * _B_PER_W

        pltpu.sync_copy(x_hbm.at[wid], idx_v)

        def start_gather(c):
            return pltpu.async_copy(
                table_hbm.at[idx_v.at[c]], rows_v.at[c % _NBUF], gsem)

        def start_store(c):
            return pltpu.async_copy(
                rows_v.at[c % _NBUF], out_hbm.at[pl.ds(base + c * _CHUNK, _CHUNK)],
                ssem)

        gathers = [start_gather(b) for b in range(_NBUF)]
        stores = []
        for c in range(_N_CHUNKS):
            gathers[c].wait()
            stores.append(start_store(c))
            nxt = c + _NBUF
            if nxt < _N_CHUNKS:
                stores[c].wait()
                gathers.append(start_gather(nxt))
        for c in range(_N_CHUNKS - _NBUF, _N_CHUNKS):
            stores[c].wait()

    return gather_kernel


_GATHER = _build()


@jax.jit
def kernel(x, table):
    idx = x.reshape(_NW, _N_CHUNKS, _CHUNK).astype(jnp.int32)
    out = _GATHER(idx, table)
    return out.reshape(_BATCH, _HIST, _EMBED)


# speedup vs baseline: 1.7651x; 1.5860x over previous
import functools

import jax
import jax.numpy as jnp
from jax import lax
from jax.experimental import pallas as pl
from jax.experimental.pallas import tpu as pltpu
from jax.experimental.pallas import tpu_sc as plsc

_BATCH = 16384
_HIST = 50
_EMBED = 32

_NC = 2
_NS = 16
_NW = _NC * _NS
_ROWS_PER_W = _BATCH // _NW
_IDX_PER_W = _ROWS_PER_W * _HIST

_CB = 16
_CHUNK = _CB * _HIST
_N_CHUNKS = _ROWS_PER_W // _CB
_NBUF = 4

_SC_PARAMS = pltpu.CompilerParams(use_tc_tiling_on_sc=False)


def _build_flatten():
    mesh = plsc.VectorSubcoreMesh(core_axis_name="c", subcore_axis_name="s")

    @functools.partial(
        pl.kernel,
        out_type=jax.ShapeDtypeStruct((_NW, _ROWS_PER_W, _HIST), jnp.int32),
        mesh=mesh,
        scratch_types=[
            pltpu.VMEM((_ROWS_PER_W, _HIST), jnp.int32),
        ],
        compiler_params=_SC_PARAMS,
    )
    def flatten_kernel(x_hbm, y_hbm, idx_v):
        wid = lax.axis_index("s") * _NC + lax.axis_index("c")
        pltpu.sync_copy(x_hbm.at[pl.ds(wid * _ROWS_PER_W, _ROWS_PER_W)], idx_v)
        pltpu.sync_copy(idx_v, y_hbm.at[wid])

    return flatten_kernel


def _build_gather():
    mesh = plsc.VectorSubcoreMesh(core_axis_name="c", subcore_axis_name="s")

    @functools.partial(
        pl.kernel,
        out_type=jax.ShapeDtypeStruct((_BATCH, _HIST, _EMBED), jnp.float32),
        mesh=mesh,
        scratch_types=[
            pltpu.VMEM((_N_CHUNKS, _CHUNK), jnp.int32),
            pltpu.VMEM((_NBUF, _CHUNK, _EMBED), jnp.float32),
            pltpu.SemaphoreType.DMA,
            pltpu.SemaphoreType.DMA,
        ],
        compiler_params=_SC_PARAMS,
    )
    def gather_kernel(y_hbm, table_hbm, out_hbm, idx_v, rows_v, gsem, ssem):
        wid = lax.axis_index("s") * _NC + lax.axis_index("c")
        base = wid * _ROWS_PER_W

        pltpu.sync_copy(y_hbm.at[wid], idx_v)

        def start_gather(c):
            return pltpu.async_copy(
                table_hbm.at[idx_v.at[c]], rows_v.at[c % _NBUF], gsem)

        def start_store(c):
            buf = rows_v.at[c % _NBUF]
            return [
                pltpu.async_copy(
                    buf.at[pl.ds(r * _HIST, _HIST)],
                    out_hbm.at[base + c * _CB + r], ssem)
                for r in range(_CB)
            ]

        def wait_stores(hs):
            for h in hs:
                h.wait()

        gathers = [start_gather(b) for b in range(_NBUF)]
        stores = []
        for c in range(_N_CHUNKS):
            gathers[c].wait()
            stores.append(start_store(c))
            nxt = c + _NBUF
            if nxt < _N_CHUNKS:
                wait_stores(stores[c])
                gathers.append(start_gather(nxt))
        for c in range(_N_CHUNKS - _NBUF, _N_CHUNKS):
            wait_stores(stores[c])

    return gather_kernel


_FLATTEN = _build_flatten()
_GATHER = _build_gather()


@jax.jit
def kernel(x, table):
    y = _FLATTEN(x.astype(jnp.int32))
    y = y.reshape(_NW, _N_CHUNKS, _CHUNK)
    return _GATHER(y, table)


# speedup vs baseline: 1.8145x; 1.0280x over previous
import functools

import jax
import jax.numpy as jnp
from jax import lax
from jax.experimental import pallas as pl
from jax.experimental.pallas import tpu as pltpu
from jax.experimental.pallas import tpu_sc as plsc

_BATCH = 16384
_HIST = 50
_EMBED = 32

_NC = 2
_NS = 16
_NW = _NC * _NS
_ROWS_PER_W = _BATCH // _NW
_NBUF = 4


def _build_gather():
    mesh = plsc.VectorSubcoreMesh(core_axis_name="c", subcore_axis_name="s")

    @functools.partial(
        pl.kernel,
        out_type=jax.ShapeDtypeStruct((_BATCH, _HIST, _EMBED), jnp.float32),
        mesh=mesh,
        scratch_types=[
            pltpu.VMEM((_HIST, _ROWS_PER_W), jnp.int32),
            pltpu.VMEM((_NBUF, _ROWS_PER_W, _EMBED), jnp.float32),
            pltpu.SemaphoreType.DMA,
            pltpu.SemaphoreType.DMA,
        ],
        compiler_params=pltpu.CompilerParams(use_tc_tiling_on_sc=False),
    )
    def gather_kernel(xt_hbm, table_hbm, out_hbm, idx_v, rows_v, gsem, ssem):
        wid = lax.axis_index("s") * _NC + lax.axis_index("c")
        base = wid * _ROWS_PER_W

        pltpu.sync_copy(xt_hbm.at[:, pl.ds(base, _ROWS_PER_W)], idx_v)

        def start_gather(h):
            return pltpu.async_copy(
                table_hbm.at[idx_v.at[h]], rows_v.at[h % _NBUF], gsem)

        def start_store(h):
            return pltpu.async_copy(
                rows_v.at[h % _NBUF],
                out_hbm.at[pl.ds(base, _ROWS_PER_W), h], ssem)

        gathers = [start_gather(b) for b in range(_NBUF)]
        stores = []
        for h in range(_HIST):
            gathers[h].wait()
            stores.append(start_store(h))
            nxt = h + _NBUF
            if nxt < _HIST:
                stores[h].wait()
                gathers.append(start_gather(nxt))
        for h in range(_HIST - _NBUF, _HIST):
            stores[h].wait()

    return gather_kernel


_GATHER = _build_gather()


@jax.jit
def kernel(x, table):
    return _GATHER(x.T.astype(jnp.int32), table)
